# baseline (speedup 1.0000x reference)
import functools

import jax
import jax.numpy as jnp
from jax import lax
from jax.experimental import pallas as pl
from jax.experimental.pallas import tpu as pltpu
from jax.experimental.pallas import tpu_sc as plsc

N_CLOTH = 8192
N_VERTS = 8192
N_FACES = 8192

WEIGHT_START = 1.0
WEIGHT_MAX = 5000.0
START_RAMPUP = 50000
N_RAMPUP = 100000
EPS = 1e-3

NC = 2
NS = 16
NW = NC * NS
L = 16

FPW = N_FACES // NW
PPW = N_CLOTH // NW

_MESH = dict(core_axis_name="c", subcore_axis_name="s", num_cores=NC,
             num_subcores=NS)
_SC_PARAMS = pltpu.CompilerParams(needs_layout_passes=False)


def _wid():
    return lax.axis_index("s") * NC + lax.axis_index("c")


def _lanes(i):
    return lax.broadcasted_iota(jnp.int32, (L,), 0) + jnp.int32(i * L)


def _stage_faces(obstacle_pos, obstacle_target_pos, obstacle_faces,
                 interpret=False):
    @functools.partial(
        pl.kernel,
        out_type=[jax.ShapeDtypeStruct((N_FACES,), jnp.float32)] * 9,
        mesh=plsc.VectorSubcoreMesh(**_MESH),
        scratch_types=[
            pltpu.VMEM((N_VERTS * 3,), jnp.float32),
            pltpu.VMEM((N_VERTS * 3,), jnp.float32),
            pltpu.VMEM((FPW * 3,), jnp.int32),
        ] + [pltpu.VMEM((FPW,), jnp.float32)] * 9,
        compiler_params=_SC_PARAMS,
        interpret=interpret,
    )
    def body(op_h, ot_h, fc_h,
             m2fx_h, m2fy_h, m2fz_h, fsq_h, nx_h, ny_h, nz_h, nsq_h, braw_h,
             op_v, ot_v, fc_v,
             m2fx_v, m2fy_v, m2fz_v, fsq_v, nx_v, ny_v, nz_v, nsq_v, braw_v):
        base = _wid() * FPW
        pltpu.sync_copy(op_h, op_v)
        pltpu.sync_copy(ot_h, ot_v)
        pltpu.sync_copy(fc_h.at[pl.ds(base * 3, FPW * 3)], fc_v)
        for i in range(FPW // L):
            sl = pl.ds(i * L, L)
            r3 = _lanes(i) * 3
            j0 = plsc.load_gather(fc_v, [r3]) * 3
            j1 = plsc.load_gather(fc_v, [r3 + 1]) * 3
            j2 = plsc.load_gather(fc_v, [r3 + 2]) * 3
            ax = plsc.load_gather(op_v, [j0])
            bx = plsc.load_gather(op_v, [j1])
            cx = plsc.load_gather(op_v, [j2])
            ay = plsc.load_gather(op_v, [j0 + 1])
            by = plsc.load_gather(op_v, [j1 + 1])
            cy = plsc.load_gather(op_v, [j2 + 1])
            az = plsc.load_gather(op_v, [j0 + 2])
            bz = plsc.load_gather(op_v, [j1 + 2])
            cz = plsc.load_gather(op_v, [j2 + 2])
            third = jnp.float32(1.0 / 3.0)
            fx = (ax + bx + cx) * third
            fy = (ay + by + cy) * third
            fz = (az + bz + cz) * third
            m2fx_v[sl] = -2.0 * fx
            m2fy_v[sl] = -2.0 * fy
            m2fz_v[sl] = -2.0 * fz
            fsq_v[sl] = fx * fx + fy * fy + fz * fz
            tax = plsc.load_gather(ot_v, [j0])
            tbx = plsc.load_gather(ot_v, [j1])
            tcx = plsc.load_gather(ot_v, [j2])
            tay = plsc.load_gather(ot_v, [j0 + 1])
            tby = plsc.load_gather(ot_v, [j1 + 1])
            tcy = plsc.load_gather(ot_v, [j2 + 1])
            taz = plsc.load_gather(ot_v, [j0 + 2])
            tbz = plsc.load_gather(ot_v, [j1 + 2])
            tcz = plsc.load_gather(ot_v, [j2 + 2])
            px = (tax + tbx + tcx) * third
            py = (tay + tby + tcy) * third
            pz = (taz + tbz + tcz) * third
            e1x = tbx - tax
            e1y = tby - tay
            e1z = tbz - taz
            e2x = tcx - tax
            e2y = tcy - tay
            e2z = tcz - taz
            nx = e1y * e2z - e1z * e2y
            ny = e1z * e2x - e1x * e2z
            nz = e1x * e2y - e1y * e2x
            nx_v[sl] = nx
            ny_v[sl] = ny
            nz_v[sl] = nz
            nsq_v[sl] = nx * nx + ny * ny + nz * nz
            braw_v[sl] = px * nx + py * ny + pz * nz
        out_sl = pl.ds(base, FPW)
        pltpu.sync_copy(m2fx_v, m2fx_h.at[out_sl])
        pltpu.sync_copy(m2fy_v, m2fy_h.at[out_sl])
        pltpu.sync_copy(m2fz_v, m2fz_h.at[out_sl])
        pltpu.sync_copy(fsq_v, fsq_h.at[out_sl])
        pltpu.sync_copy(nx_v, nx_h.at[out_sl])
        pltpu.sync_copy(ny_v, ny_h.at[out_sl])
        pltpu.sync_copy(nz_v, nz_h.at[out_sl])
        pltpu.sync_copy(nsq_v, nsq_h.at[out_sl])
        pltpu.sync_copy(braw_v, braw_h.at[out_sl])

    return body(obstacle_pos, obstacle_target_pos, obstacle_faces)


_ROWS = 512
_CHUNK = 1024


def _stage_argmin(cloth_pos, m2fx, m2fy, m2fz, fsq, interpret=False):
    def body(c_ref, fx_ref, fy_ref, fz_ref, fsq_ref, idx_ref):
        cb = c_ref[...].astype(jnp.bfloat16)
        jloc = lax.broadcasted_iota(
            jnp.int32, (_ROWS, _CHUNK), 1).astype(jnp.float32)
        acc_min = jnp.full((_ROWS, 1), jnp.inf, dtype=jnp.float32)
        acc_idx = jnp.zeros((_ROWS, 1), dtype=jnp.float32)
        for jt in range(N_FACES // _CHUNK):
            sl = pl.ds(jt * _CHUNK, _CHUNK)
            f3 = jnp.concatenate(
                [fx_ref[:, sl], fy_ref[:, sl], fz_ref[:, sl]], axis=0)
            mm = lax.dot_general(
                cb, f3.astype(jnp.bfloat16), (((1,), (0,)), ((), ())),
                preferred_element_type=jnp.float32,
            )
            s = fsq_ref[:, sl] + mm
            tmin = jnp.min(s, axis=1, keepdims=True)
            tidx = jnp.min(
                jnp.where(s <= tmin, jloc, jnp.float32(1e30)),
                axis=1, keepdims=True) + jnp.float32(jt * _CHUNK)
            upd = tmin < acc_min
            acc_idx = jnp.where(upd, tidx, acc_idx)
            acc_min = jnp.where(upd, tmin, acc_min)
        idx_ref[...] = acc_idx.astype(jnp.int32)

    return pl.pallas_call(
        body,
        grid=(N_CLOTH // _ROWS,),
        in_specs=[
            pl.BlockSpec((_ROWS, 3), lambda i: (i, 0)),
            pl.BlockSpec((1, N_FACES), lambda i: (0, 0)),
            pl.BlockSpec((1, N_FACES), lambda i: (0, 0)),
            pl.BlockSpec((1, N_FACES), lambda i: (0, 0)),
            pl.BlockSpec((1, N_FACES), lambda i: (0, 0)),
        ],
        out_specs=pl.BlockSpec((_ROWS, 1), lambda i: (i, 0)),
        out_shape=jax.ShapeDtypeStruct((N_CLOTH, 1), jnp.int32),
        interpret=interpret,
    )(cloth_pos, m2fx, m2fy, m2fz, fsq)


def _stage_loss(nn_idx, nx, ny, nz, nsq, braw, cloth_pred_pos,
                interpret=False):
    @functools.partial(
        pl.kernel,
        out_type=jax.ShapeDtypeStruct((NW, L), jnp.float32),
        mesh=plsc.VectorSubcoreMesh(**_MESH),
        scratch_types=[
            pltpu.VMEM((N_FACES,), jnp.float32),
            pltpu.VMEM((N_FACES,), jnp.float32),
            pltpu.VMEM((N_FACES,), jnp.float32),
            pltpu.VMEM((N_FACES,), jnp.float32),
            pltpu.VMEM((N_FACES,), jnp.float32),
            pltpu.VMEM((PPW,), jnp.int32),
            pltpu.VMEM((PPW * 3,), jnp.float32),
            pltpu.VMEM((L,), jnp.float32),
        ],
        compiler_params=_SC_PARAMS,
        interpret=interpret,
    )
    def body(idx_h, nx_h, ny_h, nz_h, nsq_h, braw_h, pp_h, out_h,
             nx_v, ny_v, nz_v, nsq_v, braw_v, idx_v, pp_v, acc_v):
        wid = _wid()
        base = wid * PPW
        pltpu.sync_copy(nx_h, nx_v)
        pltpu.sync_copy(ny_h, ny_v)
        pltpu.sync_copy(nz_h, nz_v)
        pltpu.sync_copy(nsq_h, nsq_v)
        pltpu.sync_copy(braw_h, braw_v)
        pltpu.sync_copy(idx_h.at[pl.ds(base, PPW)], idx_v)
        pltpu.sync_copy(pp_h.at[pl.ds(base * 3, PPW * 3)], pp_v)
        acc = jnp.zeros((L,), jnp.float32)
        for i in range(PPW // L):
            sl = pl.ds(i * L, L)
            r3 = _lanes(i) * 3
            ii = idx_v[sl]
            gx = plsc.load_gather(nx_v, [ii])
            gy = plsc.load_gather(ny_v, [ii])
            gz = plsc.load_gather(nz_v, [ii])
            gq = plsc.load_gather(nsq_v, [ii])
            gb = plsc.load_gather(braw_v, [ii])
            ppx = plsc.load_gather(pp_v, [r3])
            ppy = plsc.load_gather(pp_v, [r3 + 1])
            ppz = plsc.load_gather(pp_v, [r3 + 2])
            draw = ppx * gx + ppy * gy + ppz * gz - gb
            xc = jnp.maximum(gq, jnp.float32(1e-36))
            y = plsc.bitcast(
                jnp.int32(0x5F3759DF) - (plsc.bitcast(xc, jnp.int32) >> 1),
                jnp.float32)
            for _ in range(3):
                y = y * (1.5 - 0.5 * xc * y * y)
            snorm = gq * y
            dist = draw / (snorm + jnp.float32(1e-12))
            t = jnp.maximum(jnp.float32(EPS) - dist, 0.0)
            acc = acc + t * t * t
        acc_v[...] = acc
        pltpu.sync_copy(acc_v, out_h.at[wid])

    return body(nn_idx, nx, ny, nz, nsq, braw, cloth_pred_pos)


def kernel(cloth_pos, cloth_pred_pos, obstacle_pos, obstacle_target_pos,
           obstacle_faces, iter_num):
    m2fx, m2fy, m2fz, fsq, nx, ny, nz, nsq, braw = _stage_faces(
        obstacle_pos.reshape(-1), obstacle_target_pos.reshape(-1),
        obstacle_faces.reshape(-1))
    nn_idx = _stage_argmin(cloth_pos, m2fx[None], m2fy[None], m2fz[None],
                           fsq[None])[:, 0]
    partials = _stage_loss(nn_idx, nx, ny, nz, nsq, braw,
                           cloth_pred_pos.reshape(-1))

    it = jnp.maximum(iter_num - START_RAMPUP, 0)
    progress = jnp.minimum(it / N_RAMPUP, 1.0)
    weight = (WEIGHT_START + (WEIGHT_MAX - WEIGHT_START) * progress)
    return jnp.sum(partials) * weight.astype(jnp.float32)


# speedup vs baseline: 1.0014x; 1.0014x over previous
import functools

import jax
import jax.numpy as jnp
from jax import lax
from jax.experimental import pallas as pl
from jax.experimental.pallas import tpu as pltpu
from jax.experimental.pallas import tpu_sc as plsc

N_CLOTH = 8192
N_VERTS = 8192
N_FACES = 8192

WEIGHT_START = 1.0
WEIGHT_MAX = 5000.0
START_RAMPUP = 50000
N_RAMPUP = 100000
EPS = 1e-3

NC = 2
NS = 16
NW = NC * NS
L = 16

FPW = N_FACES // NW
PPW = N_CLOTH // NW

_MESH = dict(core_axis_name="c", subcore_axis_name="s", num_cores=NC,
             num_subcores=NS)
_SC_PARAMS = pltpu.CompilerParams(needs_layout_passes=False)


def _wid():
    return lax.axis_index("s") * NC + lax.axis_index("c")


def _lanes(i):
    return lax.broadcasted_iota(jnp.int32, (L,), 0) + jnp.int32(i * L)


def _stage_faces(obstacle_pos, obstacle_target_pos, obstacle_faces,
                 interpret=False):
    @functools.partial(
        pl.kernel,
        out_type=[jax.ShapeDtypeStruct((N_FACES,), jnp.float32)] * 9,
        mesh=plsc.VectorSubcoreMesh(**_MESH),
        scratch_types=[
            pltpu.VMEM((N_VERTS * 3,), jnp.float32),
            pltpu.VMEM((N_VERTS * 3,), jnp.float32),
            pltpu.VMEM((FPW * 3,), jnp.int32),
        ] + [pltpu.VMEM((FPW,), jnp.float32)] * 9,
        compiler_params=_SC_PARAMS,
        interpret=interpret,
    )
    def body(op_h, ot_h, fc_h,
             m2fx_h, m2fy_h, m2fz_h, fsq_h, nx_h, ny_h, nz_h, nsq_h, braw_h,
             op_v, ot_v, fc_v,
             m2fx_v, m2fy_v, m2fz_v, fsq_v, nx_v, ny_v, nz_v, nsq_v, braw_v):
        base = _wid() * FPW
        pltpu.sync_copy(op_h, op_v)
        pltpu.sync_copy(ot_h, ot_v)
        pltpu.sync_copy(fc_h.at[pl.ds(base * 3, FPW * 3)], fc_v)
        for i in range(FPW // L):
            sl = pl.ds(i * L, L)
            r3 = _lanes(i) * 3
            j0 = plsc.load_gather(fc_v, [r3]) * 3
            j1 = plsc.load_gather(fc_v, [r3 + 1]) * 3
            j2 = plsc.load_gather(fc_v, [r3 + 2]) * 3
            ax = plsc.load_gather(op_v, [j0])
            bx = plsc.load_gather(op_v, [j1])
            cx = plsc.load_gather(op_v, [j2])
            ay = plsc.load_gather(op_v, [j0 + 1])
            by = plsc.load_gather(op_v, [j1 + 1])
            cy = plsc.load_gather(op_v, [j2 + 1])
            az = plsc.load_gather(op_v, [j0 + 2])
            bz = plsc.load_gather(op_v, [j1 + 2])
            cz = plsc.load_gather(op_v, [j2 + 2])
            third = jnp.float32(1.0 / 3.0)
            fx = (ax + bx + cx) * third
            fy = (ay + by + cy) * third
            fz = (az + bz + cz) * third
            m2fx_v[sl] = -2.0 * fx
            m2fy_v[sl] = -2.0 * fy
            m2fz_v[sl] = -2.0 * fz
            fsq_v[sl] = fx * fx + fy * fy + fz * fz
            tax = plsc.load_gather(ot_v, [j0])
            tbx = plsc.load_gather(ot_v, [j1])
            tcx = plsc.load_gather(ot_v, [j2])
            tay = plsc.load_gather(ot_v, [j0 + 1])
            tby = plsc.load_gather(ot_v, [j1 + 1])
            tcy = plsc.load_gather(ot_v, [j2 + 1])
            taz = plsc.load_gather(ot_v, [j0 + 2])
            tbz = plsc.load_gather(ot_v, [j1 + 2])
            tcz = plsc.load_gather(ot_v, [j2 + 2])
            px = (tax + tbx + tcx) * third
            py = (tay + tby + tcy) * third
            pz = (taz + tbz + tcz) * third
            e1x = tbx - tax
            e1y = tby - tay
            e1z = tbz - taz
            e2x = tcx - tax
            e2y = tcy - tay
            e2z = tcz - taz
            nx = e1y * e2z - e1z * e2y
            ny = e1z * e2x - e1x * e2z
            nz = e1x * e2y - e1y * e2x
            nx_v[sl] = nx
            ny_v[sl] = ny
            nz_v[sl] = nz
            nsq_v[sl] = nx * nx + ny * ny + nz * nz
            braw_v[sl] = px * nx + py * ny + pz * nz
        out_sl = pl.ds(base, FPW)
        pltpu.sync_copy(m2fx_v, m2fx_h.at[out_sl])
        pltpu.sync_copy(m2fy_v, m2fy_h.at[out_sl])
        pltpu.sync_copy(m2fz_v, m2fz_h.at[out_sl])
        pltpu.sync_copy(fsq_v, fsq_h.at[out_sl])
        pltpu.sync_copy(nx_v, nx_h.at[out_sl])
        pltpu.sync_copy(ny_v, ny_h.at[out_sl])
        pltpu.sync_copy(nz_v, nz_h.at[out_sl])
        pltpu.sync_copy(nsq_v, nsq_h.at[out_sl])
        pltpu.sync_copy(braw_v, braw_h.at[out_sl])

    return body(obstacle_pos, obstacle_target_pos, obstacle_faces)


_ROWS = 512
_CHUNK = 1024


def _stage_argmin(cloth_pos, m2fx, m2fy, m2fz, fsq, interpret=False):
    def body(c_ref, fx_ref, fy_ref, fz_ref, fsq_ref, idx_ref):
        cb = c_ref[...].astype(jnp.bfloat16)
        f3b = jnp.concatenate(
            [fx_ref[...], fy_ref[...], fz_ref[...]],
            axis=0).astype(jnp.bfloat16)
        jloc = lax.broadcasted_iota(
            jnp.int32, (_ROWS, _CHUNK), 1).astype(jnp.float32)
        acc_min = jnp.full((_ROWS, 1), jnp.inf, dtype=jnp.float32)
        acc_idx = jnp.zeros((_ROWS, 1), dtype=jnp.float32)
        for jt in range(N_FACES // _CHUNK):
            sl = pl.ds(jt * _CHUNK, _CHUNK)
            mm = lax.dot_general(
                cb, f3b[:, jt * _CHUNK:(jt + 1) * _CHUNK],
                (((1,), (0,)), ((), ())),
                preferred_element_type=jnp.float32,
            )
            s = fsq_ref[:, sl] + mm
            tmin = jnp.min(s, axis=1, keepdims=True)
            tidx = jnp.min(
                jnp.where(s <= tmin, jloc, jnp.float32(1e30)),
                axis=1, keepdims=True) + jnp.float32(jt * _CHUNK)
            upd = tmin < acc_min
            acc_idx = jnp.where(upd, tidx, acc_idx)
            acc_min = jnp.where(upd, tmin, acc_min)
        idx_ref[...] = acc_idx.astype(jnp.int32)

    return pl.pallas_call(
        body,
        grid=(N_CLOTH // _ROWS,),
        in_specs=[
            pl.BlockSpec((_ROWS, 3), lambda i: (i, 0)),
            pl.BlockSpec((1, N_FACES), lambda i: (0, 0)),
            pl.BlockSpec((1, N_FACES), lambda i: (0, 0)),
            pl.BlockSpec((1, N_FACES), lambda i: (0, 0)),
            pl.BlockSpec((1, N_FACES), lambda i: (0, 0)),
        ],
        out_specs=pl.BlockSpec((_ROWS, 1), lambda i: (i, 0)),
        out_shape=jax.ShapeDtypeStruct((N_CLOTH, 1), jnp.int32),
        interpret=interpret,
    )(cloth_pos, m2fx, m2fy, m2fz, fsq)


def _stage_loss(nn_idx, nx, ny, nz, nsq, braw, cloth_pred_pos,
                interpret=False):
    @functools.partial(
        pl.kernel,
        out_type=jax.ShapeDtypeStruct((NW, L), jnp.float32),
        mesh=plsc.VectorSubcoreMesh(**_MESH),
        scratch_types=[
            pltpu.VMEM((N_FACES,), jnp.float32),
            pltpu.VMEM((N_FACES,), jnp.float32),
            pltpu.VMEM((N_FACES,), jnp.float32),
            pltpu.VMEM((N_FACES,), jnp.float32),
            pltpu.VMEM((N_FACES,), jnp.float32),
            pltpu.VMEM((PPW,), jnp.int32),
            pltpu.VMEM((PPW * 3,), jnp.float32),
            pltpu.VMEM((L,), jnp.float32),
        ],
        compiler_params=_SC_PARAMS,
        interpret=interpret,
    )
    def body(idx_h, nx_h, ny_h, nz_h, nsq_h, braw_h, pp_h, out_h,
             nx_v, ny_v, nz_v, nsq_v, braw_v, idx_v, pp_v, acc_v):
        wid = _wid()
        base = wid * PPW
        pltpu.sync_copy(nx_h, nx_v)
        pltpu.sync_copy(ny_h, ny_v)
        pltpu.sync_copy(nz_h, nz_v)
        pltpu.sync_copy(nsq_h, nsq_v)
        pltpu.sync_copy(braw_h, braw_v)
        pltpu.sync_copy(idx_h.at[pl.ds(base, PPW)], idx_v)
        pltpu.sync_copy(pp_h.at[pl.ds(base * 3, PPW * 3)], pp_v)
        acc = jnp.zeros((L,), jnp.float32)
        for i in range(PPW // L):
            sl = pl.ds(i * L, L)
            r3 = _lanes(i) * 3
            ii = idx_v[sl]
            gx = plsc.load_gather(nx_v, [ii])
            gy = plsc.load_gather(ny_v, [ii])
            gz = plsc.load_gather(nz_v, [ii])
            gq = plsc.load_gather(nsq_v, [ii])
            gb = plsc.load_gather(braw_v, [ii])
            ppx = plsc.load_gather(pp_v, [r3])
            ppy = plsc.load_gather(pp_v, [r3 + 1])
            ppz = plsc.load_gather(pp_v, [r3 + 2])
            draw = ppx * gx + ppy * gy + ppz * gz - gb
            xc = jnp.maximum(gq, jnp.float32(1e-36))
            y = plsc.bitcast(
                jnp.int32(0x5F3759DF) - (plsc.bitcast(xc, jnp.int32) >> 1),
                jnp.float32)
            for _ in range(3):
                y = y * (1.5 - 0.5 * xc * y * y)
            snorm = gq * y
            dist = draw / (snorm + jnp.float32(1e-12))
            t = jnp.maximum(jnp.float32(EPS) - dist, 0.0)
            acc = acc + t * t * t
        acc_v[...] = acc
        pltpu.sync_copy(acc_v, out_h.at[wid])

    return body(nn_idx, nx, ny, nz, nsq, braw, cloth_pred_pos)


def kernel(cloth_pos, cloth_pred_pos, obstacle_pos, obstacle_target_pos,
           obstacle_faces, iter_num):
    m2fx, m2fy, m2fz, fsq, nx, ny, nz, nsq, braw = _stage_faces(
        obstacle_pos.reshape(-1), obstacle_target_pos.reshape(-1),
        obstacle_faces.reshape(-1))
    nn_idx = _stage_argmin(cloth_pos, m2fx[None], m2fy[None], m2fz[None],
                           fsq[None])[:, 0]
    partials = _stage_loss(nn_idx, nx, ny, nz, nsq, braw,
                           cloth_pred_pos.reshape(-1))

    it = jnp.maximum(iter_num - START_RAMPUP, 0)
    progress = jnp.minimum(it / N_RAMPUP, 1.0)
    weight = (WEIGHT_START + (WEIGHT_MAX - WEIGHT_START) * progress)
    return jnp.sum(partials) * weight.astype(jnp.float32)
